# baseline (speedup 1.0000x reference)
import jax
import jax.numpy as jnp
from jax.experimental import pallas as pl
from jax.experimental.pallas import tpu as pltpu

EMBED_DIM = 128
SPATIAL = 32 * 32


def _broadcast_body(t_ref, emb_ref, o_ref):
    e = emb_ref[0, 0, :]
    o_ref[...] = jnp.broadcast_to(e[None, :, None], (1, EMBED_DIM, SPATIAL))


def kernel(x, t, embedding):
    B = t.shape[0]
    emb3 = embedding.reshape(embedding.shape[0], 1, EMBED_DIM)
    grid_spec = pltpu.PrefetchScalarGridSpec(
        num_scalar_prefetch=1,
        grid=(B,),
        in_specs=[
            pl.BlockSpec((1, 1, EMBED_DIM), lambda i, t_ref: (t_ref[i], 0, 0)),
        ],
        out_specs=pl.BlockSpec((1, EMBED_DIM, SPATIAL), lambda i, t_ref: (i, 0, 0)),
    )
    out = pl.pallas_call(
        _broadcast_body,
        grid_spec=grid_spec,
        out_shape=jax.ShapeDtypeStruct((B, EMBED_DIM, SPATIAL), jnp.float32),
    )(t, emb3)
    return out.reshape(B, EMBED_DIM, x.shape[-2], x.shape[-1])


# speedup vs baseline: 1.6127x; 1.6127x over previous
import jax
import jax.numpy as jnp
from jax.experimental import pallas as pl
from jax.experimental.pallas import tpu as pltpu

EMBED_DIM = 128
SPATIAL = 32 * 32
BB = 8


def _gather_body(emb_ref, t_ref, g_ref):
    vpad = emb_ref.shape[0]
    b = t_ref.shape[0]
    cols = jax.lax.broadcasted_iota(jnp.int32, (b, vpad), 1)
    onehot = (cols == t_ref[:, 0][:, None]).astype(jnp.float32)
    g_ref[...] = jax.lax.dot_general(
        onehot, emb_ref[...], (((1,), (0,)), ((), ())),
        preferred_element_type=jnp.float32)


def _broadcast_body(g_ref, o_ref):
    gt = jnp.swapaxes(g_ref[...], 0, 1)
    for j in range(BB):
        o_ref[j] = jnp.broadcast_to(gt[:, j:j + 1], (EMBED_DIM, SPATIAL))


def kernel(x, t, embedding):
    B = t.shape[0]
    V = embedding.shape[0]
    vpad = (V + 7) // 8 * 8
    emb_pad = jnp.pad(embedding, ((0, vpad - V), (0, 0)))

    g = pl.pallas_call(
        _gather_body,
        out_shape=jax.ShapeDtypeStruct((B, EMBED_DIM), jnp.float32),
    )(emb_pad, t.reshape(B, 1))

    out = pl.pallas_call(
        _broadcast_body,
        grid=(B // BB,),
        in_specs=[pl.BlockSpec((BB, EMBED_DIM), lambda i: (i, 0))],
        out_specs=pl.BlockSpec((BB, EMBED_DIM, SPATIAL), lambda i: (i, 0, 0)),
        out_shape=jax.ShapeDtypeStruct((B, EMBED_DIM, SPATIAL), jnp.float32),
    )(g)
    return out.reshape(B, EMBED_DIM, x.shape[-2], x.shape[-1])
